# baseline (speedup 1.0000x reference)
import math

import jax
import jax.numpy as jnp
from jax import lax
from jax.experimental import pallas as pl
from jax.experimental.pallas import tpu as pltpu
from jax.experimental.pallas import tpu_sc as plsc

VOCAB = 1000000
D = 64
B = 4096 * 200
SCALE = math.sqrt(D)

_info = plsc.get_sparse_core_info()
NC, NS, L = _info.num_cores, _info.num_subcores, _info.num_lanes
NW = NC * NS
B_PER_W = B // NW
CHUNK = 128
N_CHUNKS = B_PER_W // CHUNK


def _body(idx_hbm, table_hbm, out_hbm, idx_v, rows_v, sem):
    wid = lax.axis_index("s") * NC + lax.axis_index("c")
    base = wid * B_PER_W

    pltpu.sync_copy(idx_hbm.at[pl.ds(wid * N_CHUNKS, N_CHUNKS)], idx_v)

    def chunk_step(j, _):
        pltpu.async_copy(table_hbm.at[idx_v.at[j]], rows_v, sem).wait()

        def scale_row(r, _):
            for c in range(D // L):
                sl = pl.ds(c * L, L)
                rows_v[r, sl] = rows_v[r, sl] * SCALE
            return ()

        lax.fori_loop(0, CHUNK, scale_row, ())
        pltpu.sync_copy(rows_v, out_hbm.at[pl.ds(base + j * CHUNK, CHUNK)])
        return ()

    lax.fori_loop(0, N_CHUNKS, chunk_step, ())


@jax.jit
def _embed(idx2d, table):
    mesh = plsc.VectorSubcoreMesh(core_axis_name="c", subcore_axis_name="s")
    f = pl.kernel(
        _body,
        out_type=jax.ShapeDtypeStruct((B, D), jnp.float32),
        mesh=mesh,
        scratch_types=[
            pltpu.VMEM((N_CHUNKS, CHUNK), jnp.int32),
            pltpu.VMEM((CHUNK, D), jnp.float32),
            pltpu.SemaphoreType.DMA,
        ],
        compiler_params=pltpu.CompilerParams(use_tc_tiling_on_sc=False),
    )
    return f(idx2d, table)


def kernel(x, embed_weight):
    idx2d = x.astype(jnp.int32).reshape(B // CHUNK, CHUNK)
    out = _embed(idx2d, table=embed_weight)
    return out.reshape(x.shape + (D,))


# speedup vs baseline: 1.2098x; 1.2098x over previous
import math

import jax
import jax.numpy as jnp
from jax import lax
from jax.experimental import pallas as pl
from jax.experimental.pallas import tpu as pltpu
from jax.experimental.pallas import tpu_sc as plsc

VOCAB = 1000000
D = 64
B = 4096 * 200
SCALE = math.sqrt(D)

_info = plsc.get_sparse_core_info()
NC, NS, L = _info.num_cores, _info.num_subcores, _info.num_lanes
NW = NC * NS
B_PER_W = B // NW
CHUNK = 128
N_CHUNKS = B_PER_W // CHUNK
NB = 8
AHEAD = 4
GROUPS = N_CHUNKS // NB


def _body(idx_hbm, table_hbm, out_hbm, idx_v, rows_v, sem_g, sem_s):
    wid = lax.axis_index("s") * NC + lax.axis_index("c")
    base = wid * B_PER_W

    pltpu.sync_copy(idx_hbm.at[pl.ds(wid * N_CHUNKS, N_CHUNKS)], idx_v)

    def fire_gather(t, bt):
        pltpu.async_copy(table_hbm.at[idx_v.at[t]], rows_v.at[bt], sem_g.at[bt])

    for c in range(AHEAD):
        fire_gather(c, c)

    def group(gi, _):
        for k in range(NB):
            c = gi * NB + k
            b = k
            bt = (k + AHEAD) % NB
            t = c + AHEAD

            @pl.when(t < N_CHUNKS)
            def _():
                @pl.when(t >= NB)
                def _():
                    pltpu.make_async_copy(
                        out_hbm.at[pl.ds(0, CHUNK)], rows_v.at[bt], sem_s.at[bt]
                    ).wait()

                fire_gather(t, bt)

            pltpu.make_async_copy(
                table_hbm.at[pl.ds(0, CHUNK)], rows_v.at[b], sem_g.at[b]
            ).wait()

            def scale_rows(r2, _):
                r = r2 * 2
                for rr in range(2):
                    for col in range(D // L):
                        sl = pl.ds(col * L, L)
                        rows_v[b, r + rr, sl] = rows_v[b, r + rr, sl] * SCALE
                return ()

            lax.fori_loop(0, CHUNK // 2, scale_rows, ())
            pltpu.async_copy(
                rows_v.at[b], out_hbm.at[pl.ds(base + c * CHUNK, CHUNK)], sem_s.at[b]
            )
        return ()

    lax.fori_loop(0, GROUPS, group, ())

    for b in range(NB):
        pltpu.make_async_copy(
            out_hbm.at[pl.ds(0, CHUNK)], rows_v.at[b], sem_s.at[b]
        ).wait()


@jax.jit
def _embed(idx2d, table):
    mesh = plsc.VectorSubcoreMesh(core_axis_name="c", subcore_axis_name="s")
    f = pl.kernel(
        _body,
        out_type=jax.ShapeDtypeStruct((B, D), jnp.float32),
        mesh=mesh,
        scratch_types=[
            pltpu.VMEM((N_CHUNKS, CHUNK), jnp.int32),
            pltpu.VMEM((NB, CHUNK, D), jnp.float32),
            pltpu.SemaphoreType.DMA((NB,)),
            pltpu.SemaphoreType.DMA((NB,)),
        ],
        compiler_params=pltpu.CompilerParams(use_tc_tiling_on_sc=False),
    )
    return f(idx2d, table)


def kernel(x, embed_weight):
    idx2d = x.astype(jnp.int32).reshape(B // CHUNK, CHUNK)
    out = _embed(idx2d, table=embed_weight)
    return out.reshape(x.shape + (D,))
